# Initial kernel scaffold; baseline (speedup 1.0000x reference)
import jax
import jax.numpy as jnp
from jax.experimental import pallas as pl


def kernel(x, edge_index, y, train_mask, W1_l, b1_l, W1_r, b1_r, W2_l, b2_l, W2_r, b2_r):
    raise NotImplementedError("write your pallas kernel here")


# speedup vs baseline: 5.5126x; 5.5126x over previous
import functools

import jax
import jax.numpy as jnp
from jax import lax
from jax.experimental import pallas as pl
from jax.experimental.pallas import tpu as pltpu
from jax.experimental.pallas import tpu_sc as plsc

N_NODES = 10000
N_EDGES = 320000
D_IN = 128
D_HID = 128
D_OUT = 64

NC = 2
NS = 16
NW = NC * NS
E_PER_TILE = N_EDGES // NW
CHUNK = 80
N_CHUNKS = E_PER_TILE // CHUNK
N_PAD = 10240
ROWS_PER_TILE = N_PAD // NS

D1P = D_HID + 16
D2P = D_OUT + 8

_sc_mesh = plsc.VectorSubcoreMesh(core_axis_name="c", subcore_axis_name="s")


def _make_sc_agg(d):
    @functools.partial(
        pl.kernel,
        mesh=_sc_mesh,
        compiler_params=pltpu.CompilerParams(use_tc_tiling_on_sc=False),
        out_type=jax.ShapeDtypeStruct((NC, N_PAD, d), jnp.float32),
        scratch_types=[
            pltpu.VMEM((CHUNK,), jnp.int32),
            pltpu.VMEM((CHUNK,), jnp.int32),
            pltpu.VMEM((CHUNK, d), jnp.float32),
            pltpu.VMEM_SHARED((N_PAD, d), jnp.float32),
            pltpu.SemaphoreType.DMA,
        ],
    )
    def sc_agg(z_hbm, src_hbm, dst_hbm, zinit_hbm, out_hbm,
               src_v, dst_v, rows_v, acc_sh, sem):
        cid = lax.axis_index("c")
        sid = lax.axis_index("s")
        wid = cid * NS + sid

        pltpu.sync_copy(zinit_hbm,
                        acc_sh.at[pl.ds(sid * ROWS_PER_TILE, ROWS_PER_TILE), :])
        plsc.subcore_barrier()

        def body(j, carry):
            off = pl.multiple_of(wid * E_PER_TILE + j * CHUNK, CHUNK)
            pltpu.sync_copy(src_hbm.at[pl.ds(off, CHUNK)], src_v)
            pltpu.sync_copy(dst_hbm.at[pl.ds(off, CHUNK)], dst_v)
            pltpu.async_copy(z_hbm.at[src_v], rows_v, sem).wait()
            pltpu.sync_copy(rows_v, acc_sh.at[dst_v], add=True)
            return carry

        lax.fori_loop(0, N_CHUNKS, body, 0)
        plsc.subcore_barrier()

        pltpu.sync_copy(acc_sh.at[pl.ds(sid * ROWS_PER_TILE, ROWS_PER_TILE), :],
                        out_hbm.at[cid, pl.ds(sid * ROWS_PER_TILE, ROWS_PER_TILE), :])

    return sc_agg


_sc_agg1 = _make_sc_agg(D1P)
_sc_agg2 = _make_sc_agg(D_OUT)


_RB = 1000
_NRB = N_NODES // _RB


def _tc1_body(x_ref, w1l_ref, w1r_ref, b1r_ref, z1p_ref, r1_ref):
    x = x_ref[...]
    z1 = jnp.dot(x, w1l_ref[...], preferred_element_type=jnp.float32)
    ones = jnp.ones((_RB, 1), jnp.float32)
    pad = jnp.zeros((_RB, D1P - D_HID - 1), jnp.float32)
    z1p_ref[...] = jnp.concatenate([z1, ones, pad], axis=1)
    r1_ref[...] = (jnp.dot(x, w1r_ref[...], preferred_element_type=jnp.float32)
                   + b1r_ref[...])


def _tc1(x, w1l, w1r, b1r):
    return pl.pallas_call(
        _tc1_body,
        grid=(_NRB,),
        in_specs=[
            pl.BlockSpec((_RB, D_IN), lambda i: (i, 0)),
            pl.BlockSpec((D_IN, D_HID), lambda i: (0, 0)),
            pl.BlockSpec((D_IN, D_HID), lambda i: (0, 0)),
            pl.BlockSpec((1, D_HID), lambda i: (0, 0)),
        ],
        out_specs=[
            pl.BlockSpec((_RB, D1P), lambda i: (i, 0)),
            pl.BlockSpec((_RB, D_HID), lambda i: (i, 0)),
        ],
        out_shape=[
            jax.ShapeDtypeStruct((N_NODES, D1P), jnp.float32),
            jax.ShapeDtypeStruct((N_NODES, D_HID), jnp.float32),
        ],
    )(x, w1l, w1r, b1r)


def _tc2_body(p1_ref, r1_ref, b1l_ref, w2l_ref, w2r_ref, b2c_ref,
              z2_ref, r2p_ref):
    s = p1_ref[0] + p1_ref[1]
    agg = s[:, :D_HID]
    deg = s[:, D_HID:D_HID + 1]
    invd = 1.0 / jnp.maximum(deg, 1.0)
    h = jnp.maximum(agg * invd + b1l_ref[...] + r1_ref[...], 0.0)
    z2_ref[...] = jnp.dot(h, w2l_ref[...], preferred_element_type=jnp.float32)
    r2c = (jnp.dot(h, w2r_ref[...], preferred_element_type=jnp.float32)
           + b2c_ref[...])
    pad = jnp.zeros((_RB, D2P - D_OUT - 1), jnp.float32)
    r2p_ref[...] = jnp.concatenate([r2c, invd, pad], axis=1)


def _tc2(p1, r1, b1l, w2l, w2r, b2c):
    return pl.pallas_call(
        _tc2_body,
        grid=(_NRB,),
        in_specs=[
            pl.BlockSpec((NC, _RB, D1P), lambda i: (0, i, 0)),
            pl.BlockSpec((_RB, D_HID), lambda i: (i, 0)),
            pl.BlockSpec((1, D_HID), lambda i: (0, 0)),
            pl.BlockSpec((D_HID, D_OUT), lambda i: (0, 0)),
            pl.BlockSpec((D_HID, D_OUT), lambda i: (0, 0)),
            pl.BlockSpec((1, D_OUT), lambda i: (0, 0)),
        ],
        out_specs=[
            pl.BlockSpec((_RB, D_OUT), lambda i: (i, 0)),
            pl.BlockSpec((_RB, D2P), lambda i: (i, 0)),
        ],
        out_shape=[
            jax.ShapeDtypeStruct((N_NODES, D_OUT), jnp.float32),
            jax.ShapeDtypeStruct((N_NODES, D2P), jnp.float32),
        ],
    )(p1, r1, b1l, w2l, w2r, b2c)


def _tc3_body(p2_ref, r2p_ref, y_ref, m_ref, out_ref, num_ref, den_ref):
    i = pl.program_id(0)

    agg2 = p2_ref[0] + p2_ref[1]
    r2c = r2p_ref[:, :D_OUT]
    invd = r2p_ref[:, D_OUT:D_OUT + 1]
    logits = agg2 * invd + r2c
    mx = jnp.max(logits, axis=1, keepdims=True)
    lse = jnp.log(jnp.sum(jnp.exp(logits - mx), axis=1, keepdims=True))
    lsm = logits - mx - lse
    onehot = (lax.broadcasted_iota(jnp.int32, (_RB, D_OUT), 1)
              == y_ref[...]).astype(jnp.float32)
    picked = jnp.sum(lsm * onehot, axis=1, keepdims=True)
    m = m_ref[...]
    num_p = jnp.sum(picked * m)
    den_p = jnp.sum(m)

    @pl.when(i == 0)
    def _():
        num_ref[0] = num_p
        den_ref[0] = den_p

    @pl.when(i > 0)
    def _():
        num_ref[0] = num_ref[0] + num_p
        den_ref[0] = den_ref[0] + den_p

    @pl.when(i == _NRB - 1)
    def _():
        loss = -num_ref[0] / jnp.maximum(den_ref[0], 1.0)
        out_ref[...] = jnp.broadcast_to(loss, (1, 1))


def _tc3(p2, r2p, y2d, m2d):
    return pl.pallas_call(
        _tc3_body,
        grid=(_NRB,),
        in_specs=[
            pl.BlockSpec((NC, _RB, D_OUT), lambda i: (0, i, 0)),
            pl.BlockSpec((_RB, D2P), lambda i: (i, 0)),
            pl.BlockSpec((_RB, 1), lambda i: (i, 0)),
            pl.BlockSpec((_RB, 1), lambda i: (i, 0)),
        ],
        out_specs=pl.BlockSpec((1, 1), lambda i: (0, 0)),
        out_shape=jax.ShapeDtypeStruct((1, 1), jnp.float32),
        scratch_shapes=[
            pltpu.SMEM((1,), jnp.float32),
            pltpu.SMEM((1,), jnp.float32),
        ],
    )(p2, r2p, y2d, m2d)


def kernel(x, edge_index, y, train_mask,
           W1_l, b1_l, W1_r, b1_r, W2_l, b2_l, W2_r, b2_r):
    src = edge_index[0]
    dst = edge_index[1]
    zinit1 = jnp.zeros((ROWS_PER_TILE, D1P), jnp.float32)
    zinit2 = jnp.zeros((ROWS_PER_TILE, D_OUT), jnp.float32)

    z1p, r1 = _tc1(x, W1_l, W1_r, b1_r.reshape(1, D_HID))
    p1 = _sc_agg1(z1p, src, dst, zinit1)
    b2c = (b2_l + b2_r).reshape(1, D_OUT)
    z2, r2p = _tc2(p1, r1, b1_l.reshape(1, D_HID), W2_l, W2_r, b2c)
    p2 = _sc_agg2(z2, src, dst, zinit2)
    loss = _tc3(p2, r2p, y.reshape(N_NODES, 1).astype(jnp.int32),
                train_mask.reshape(N_NODES, 1).astype(jnp.float32))
    return loss.reshape(1)


# speedup vs baseline: 13.9978x; 2.5392x over previous
import functools

import jax
import jax.numpy as jnp
from jax import lax
from jax.experimental import pallas as pl
from jax.experimental.pallas import tpu as pltpu
from jax.experimental.pallas import tpu_sc as plsc

N_NODES = 10000
N_EDGES = 320000
D_IN = 128
D_HID = 128
D_OUT = 64

NC = 2
NS = 16
NW = NC * NS
E_PER_TILE = N_EDGES // NW
N_PAD = 10240
ROWS_PER_TILE = N_PAD // NS

D2P = D_OUT + 8
DDEG = 8

NBUF = 5

_sc_mesh = plsc.VectorSubcoreMesh(core_axis_name="c", subcore_axis_name="s")
_sc_params = pltpu.CompilerParams(use_tc_tiling_on_sc=False)


def _make_sc_agg(d, chunk):
    n_chunks = E_PER_TILE // chunk
    n_groups = n_chunks // NBUF

    @functools.partial(
        pl.kernel,
        mesh=_sc_mesh,
        compiler_params=_sc_params,
        out_type=jax.ShapeDtypeStruct((NC, N_PAD, d), jnp.float32),
        scratch_types=[
            pltpu.VMEM((n_chunks, chunk), jnp.int32),
            pltpu.VMEM((n_chunks, chunk), jnp.int32),
            pltpu.VMEM((NBUF, chunk, d), jnp.float32),
            pltpu.VMEM_SHARED((N_PAD, d), jnp.float32),
            pltpu.SemaphoreType.DMA((NBUF,)),
            pltpu.SemaphoreType.DMA((NBUF,)),
        ],
    )
    def sc_agg(z_hbm, src_hbm, dst_hbm, zinit_hbm, out_hbm,
               src_v, dst_v, rows_v, acc_sh, gsem, ssem):
        cid = lax.axis_index("c")
        sid = lax.axis_index("s")
        wid = cid * NS + sid

        pltpu.sync_copy(src_hbm.at[wid], src_v)
        pltpu.sync_copy(dst_hbm.at[wid], dst_v)
        pltpu.sync_copy(zinit_hbm,
                        acc_sh.at[pl.ds(sid * ROWS_PER_TILE, ROWS_PER_TILE), :])
        plsc.subcore_barrier()

        for t in range(NBUF):
            pltpu.async_copy(z_hbm.at[src_v.at[t]], rows_v.at[t], gsem.at[t])

        def body(g, carry):
            j0 = g * NBUF
            for t in range(NBUF):
                j = j0 + t
                pltpu.make_async_copy(z_hbm.at[src_v.at[j]], rows_v.at[t],
                                      gsem.at[t]).wait()
                pltpu.async_copy(rows_v.at[t], acc_sh.at[dst_v.at[j]],
                                 ssem.at[t], add=True)
            for t in range(NBUF):
                j = j0 + t
                jn = j + NBUF
                pltpu.make_async_copy(rows_v.at[t], acc_sh.at[dst_v.at[j]],
                                      ssem.at[t]).wait()

                @pl.when(jn < n_chunks)
                def _():
                    pltpu.async_copy(z_hbm.at[src_v.at[jn]], rows_v.at[t],
                                     gsem.at[t])
            return carry

        lax.fori_loop(0, n_groups, body, 0)
        plsc.subcore_barrier()

        pltpu.sync_copy(acc_sh.at[pl.ds(sid * ROWS_PER_TILE, ROWS_PER_TILE), :],
                        out_hbm.at[cid, pl.ds(sid * ROWS_PER_TILE, ROWS_PER_TILE), :])

    return sc_agg


_C1 = 40
_C2 = 80
_sc_agg1 = _make_sc_agg(D_HID, _C1)
_sc_agg2 = _make_sc_agg(D_OUT, _C2)

_CD = 80
_ND_CHUNKS = E_PER_TILE // _CD
_ND_GROUPS = _ND_CHUNKS // NBUF


@functools.partial(
    pl.kernel,
    mesh=_sc_mesh,
    compiler_params=_sc_params,
    out_type=jax.ShapeDtypeStruct((NC, N_PAD, DDEG), jnp.float32),
    scratch_types=[
        pltpu.VMEM((_ND_CHUNKS, _CD), jnp.int32),
        pltpu.VMEM((_CD, DDEG), jnp.float32),
        pltpu.VMEM_SHARED((N_PAD, DDEG), jnp.float32),
        pltpu.SemaphoreType.DMA((NBUF,)),
    ],
)
def _sc_deg(dst_hbm, ones_hbm, zinit_hbm, out_hbm, dst_v, ones_v, acc_sh, ssem):
    cid = lax.axis_index("c")
    sid = lax.axis_index("s")
    wid = cid * NS + sid

    pltpu.sync_copy(dst_hbm.at[wid], dst_v)
    pltpu.sync_copy(ones_hbm, ones_v)
    pltpu.sync_copy(zinit_hbm,
                    acc_sh.at[pl.ds(sid * ROWS_PER_TILE, ROWS_PER_TILE), :])
    plsc.subcore_barrier()

    def body(g, carry):
        j0 = g * NBUF
        for t in range(NBUF):
            pltpu.async_copy(ones_v, acc_sh.at[dst_v.at[j0 + t]],
                             ssem.at[t], add=True)
        for t in range(NBUF):
            pltpu.make_async_copy(ones_v, acc_sh.at[dst_v.at[j0 + t]],
                                  ssem.at[t]).wait()
        return carry

    lax.fori_loop(0, _ND_GROUPS, body, 0)
    plsc.subcore_barrier()

    pltpu.sync_copy(acc_sh.at[pl.ds(sid * ROWS_PER_TILE, ROWS_PER_TILE), :],
                    out_hbm.at[cid, pl.ds(sid * ROWS_PER_TILE, ROWS_PER_TILE), :])


_RB = 1000
_NRB = N_NODES // _RB


def _tc1_body(x_ref, w1l_ref, w1r_ref, b1r_ref, z1_ref, r1_ref):
    x = x_ref[...]
    z1_ref[...] = jnp.dot(x, w1l_ref[...], preferred_element_type=jnp.float32)
    r1_ref[...] = (jnp.dot(x, w1r_ref[...], preferred_element_type=jnp.float32)
                   + b1r_ref[...])


def _tc1(x, w1l, w1r, b1r):
    return pl.pallas_call(
        _tc1_body,
        grid=(_NRB,),
        in_specs=[
            pl.BlockSpec((_RB, D_IN), lambda i: (i, 0)),
            pl.BlockSpec((D_IN, D_HID), lambda i: (0, 0)),
            pl.BlockSpec((D_IN, D_HID), lambda i: (0, 0)),
            pl.BlockSpec((1, D_HID), lambda i: (0, 0)),
        ],
        out_specs=[
            pl.BlockSpec((_RB, D_HID), lambda i: (i, 0)),
            pl.BlockSpec((_RB, D_HID), lambda i: (i, 0)),
        ],
        out_shape=[
            jax.ShapeDtypeStruct((N_NODES, D_HID), jnp.float32),
            jax.ShapeDtypeStruct((N_NODES, D_HID), jnp.float32),
        ],
    )(x, w1l, w1r, b1r)


def _tc2_body(p1_ref, ds_ref, r1_ref, b1l_ref, w2l_ref, w2r_ref, b2c_ref,
              z2_ref, r2p_ref):
    agg = p1_ref[0] + p1_ref[1]
    ds = ds_ref[0] + ds_ref[1]
    deg = ds[:, 0:1]
    invd = 1.0 / jnp.maximum(deg, 1.0)
    h = jnp.maximum(agg * invd + b1l_ref[...] + r1_ref[...], 0.0)
    z2_ref[...] = jnp.dot(h, w2l_ref[...], preferred_element_type=jnp.float32)
    r2c = (jnp.dot(h, w2r_ref[...], preferred_element_type=jnp.float32)
           + b2c_ref[...])
    pad = jnp.zeros((_RB, D2P - D_OUT - 1), jnp.float32)
    r2p_ref[...] = jnp.concatenate([r2c, invd, pad], axis=1)


def _tc2(p1, dsum, r1, b1l, w2l, w2r, b2c):
    return pl.pallas_call(
        _tc2_body,
        grid=(_NRB,),
        in_specs=[
            pl.BlockSpec((NC, _RB, D_HID), lambda i: (0, i, 0)),
            pl.BlockSpec((NC, _RB, DDEG), lambda i: (0, i, 0)),
            pl.BlockSpec((_RB, D_HID), lambda i: (i, 0)),
            pl.BlockSpec((1, D_HID), lambda i: (0, 0)),
            pl.BlockSpec((D_HID, D_OUT), lambda i: (0, 0)),
            pl.BlockSpec((D_HID, D_OUT), lambda i: (0, 0)),
            pl.BlockSpec((1, D_OUT), lambda i: (0, 0)),
        ],
        out_specs=[
            pl.BlockSpec((_RB, D_OUT), lambda i: (i, 0)),
            pl.BlockSpec((_RB, D2P), lambda i: (i, 0)),
        ],
        out_shape=[
            jax.ShapeDtypeStruct((N_NODES, D_OUT), jnp.float32),
            jax.ShapeDtypeStruct((N_NODES, D2P), jnp.float32),
        ],
    )(p1, dsum, r1, b1l, w2l, w2r, b2c)


def _tc3_body(p2_ref, r2p_ref, y_ref, m_ref, out_ref, num_ref, den_ref):
    i = pl.program_id(0)

    agg2 = p2_ref[0] + p2_ref[1]
    r2c = r2p_ref[:, :D_OUT]
    invd = r2p_ref[:, D_OUT:D_OUT + 1]
    logits = agg2 * invd + r2c
    mx = jnp.max(logits, axis=1, keepdims=True)
    lse = jnp.log(jnp.sum(jnp.exp(logits - mx), axis=1, keepdims=True))
    lsm = logits - mx - lse
    onehot = (lax.broadcasted_iota(jnp.int32, (_RB, D_OUT), 1)
              == y_ref[...]).astype(jnp.float32)
    picked = jnp.sum(lsm * onehot, axis=1, keepdims=True)
    m = m_ref[...]
    num_p = jnp.sum(picked * m)
    den_p = jnp.sum(m)

    @pl.when(i == 0)
    def _():
        num_ref[0] = num_p
        den_ref[0] = den_p

    @pl.when(i > 0)
    def _():
        num_ref[0] = num_ref[0] + num_p
        den_ref[0] = den_ref[0] + den_p

    @pl.when(i == _NRB - 1)
    def _():
        loss = -num_ref[0] / jnp.maximum(den_ref[0], 1.0)
        out_ref[...] = jnp.broadcast_to(loss, (1, 1))


def _tc3(p2, r2p, y2d, m2d):
    return pl.pallas_call(
        _tc3_body,
        grid=(_NRB,),
        in_specs=[
            pl.BlockSpec((NC, _RB, D_OUT), lambda i: (0, i, 0)),
            pl.BlockSpec((_RB, D2P), lambda i: (i, 0)),
            pl.BlockSpec((_RB, 1), lambda i: (i, 0)),
            pl.BlockSpec((_RB, 1), lambda i: (i, 0)),
        ],
        out_specs=pl.BlockSpec((1, 1), lambda i: (0, 0)),
        out_shape=jax.ShapeDtypeStruct((1, 1), jnp.float32),
        scratch_shapes=[
            pltpu.SMEM((1,), jnp.float32),
            pltpu.SMEM((1,), jnp.float32),
        ],
    )(p2, r2p, y2d, m2d)


def kernel(x, edge_index, y, train_mask,
           W1_l, b1_l, W1_r, b1_r, W2_l, b2_l, W2_r, b2_r):
    src = edge_index[0]
    dst = edge_index[1]
    src1 = src.reshape(NW, E_PER_TILE // _C1, _C1)
    dst1 = dst.reshape(NW, E_PER_TILE // _C1, _C1)
    src2 = src.reshape(NW, E_PER_TILE // _C2, _C2)
    dst2 = dst.reshape(NW, E_PER_TILE // _C2, _C2)
    zinit1 = jnp.zeros((ROWS_PER_TILE, D_HID), jnp.float32)
    zinit2 = jnp.zeros((ROWS_PER_TILE, D_OUT), jnp.float32)
    zinitd = jnp.zeros((ROWS_PER_TILE, DDEG), jnp.float32)
    onesd = jnp.ones((_CD, DDEG), jnp.float32)

    dsum = _sc_deg(dst2, onesd, zinitd)
    z1, r1 = _tc1(x, W1_l, W1_r, b1_r.reshape(1, D_HID))
    p1 = _sc_agg1(z1, src1, dst1, zinit1)
    b2c = (b2_l + b2_r).reshape(1, D_OUT)
    z2, r2p = _tc2(p1, dsum, r1, b1_l.reshape(1, D_HID), W2_l, W2_r, b2c)
    p2 = _sc_agg2(z2, src2, dst2, zinit2)
    loss = _tc3(p2, r2p, y.reshape(N_NODES, 1).astype(jnp.int32),
                train_mask.reshape(N_NODES, 1).astype(jnp.float32))
    return loss.reshape(1)


# speedup vs baseline: 16.3335x; 1.1669x over previous
import functools

import jax
import jax.numpy as jnp
from jax import lax
from jax.experimental import pallas as pl
from jax.experimental.pallas import tpu as pltpu
from jax.experimental.pallas import tpu_sc as plsc

N_NODES = 10000
N_EDGES = 320000
D_IN = 128
D_HID = 128
D_OUT = 64

NC = 2
NS = 16
NW = NC * NS
E_PER_TILE = N_EDGES // NW
N_PAD = 10240
ROWS_PER_TILE = N_PAD // NS

D2P = D_OUT + 8
DDEG = 8

NBUF = 5

_sc_mesh = plsc.VectorSubcoreMesh(core_axis_name="c", subcore_axis_name="s")
_sc_params = pltpu.CompilerParams(use_tc_tiling_on_sc=False)


def _make_sc_agg(d, chunk):
    n_chunks = E_PER_TILE // chunk
    n_groups = n_chunks // NBUF

    @functools.partial(
        pl.kernel,
        mesh=_sc_mesh,
        compiler_params=_sc_params,
        out_type=jax.ShapeDtypeStruct((NC, N_PAD, d), jnp.bfloat16),
        scratch_types=[
            pltpu.VMEM((n_chunks, chunk), jnp.int32),
            pltpu.VMEM((n_chunks, chunk), jnp.int32),
            pltpu.VMEM((NBUF, chunk, d), jnp.bfloat16),
            pltpu.VMEM_SHARED((N_PAD, d), jnp.bfloat16),
            pltpu.SemaphoreType.DMA((NBUF,)),
            pltpu.SemaphoreType.DMA((NBUF,)),
        ],
    )
    def sc_agg(z_hbm, src_hbm, dst_hbm, zinit_hbm, out_hbm,
               src_v, dst_v, rows_v, acc_sh, gsem, ssem):
        cid = lax.axis_index("c")
        sid = lax.axis_index("s")
        wid = cid * NS + sid

        pltpu.sync_copy(src_hbm.at[wid], src_v)
        pltpu.sync_copy(dst_hbm.at[wid], dst_v)
        pltpu.sync_copy(zinit_hbm,
                        acc_sh.at[pl.ds(sid * ROWS_PER_TILE, ROWS_PER_TILE), :])
        plsc.subcore_barrier()

        for t in range(NBUF):
            pltpu.async_copy(z_hbm.at[src_v.at[t]], rows_v.at[t], gsem.at[t])

        def body(g, carry):
            j0 = g * NBUF
            for t in range(NBUF):
                j = j0 + t
                pltpu.make_async_copy(z_hbm.at[src_v.at[j]], rows_v.at[t],
                                      gsem.at[t]).wait()
                pltpu.async_copy(rows_v.at[t], acc_sh.at[dst_v.at[j]],
                                 ssem.at[t], add=True)
            for t in range(NBUF):
                j = j0 + t
                jn = j + NBUF
                pltpu.make_async_copy(rows_v.at[t], acc_sh.at[dst_v.at[j]],
                                      ssem.at[t]).wait()

                @pl.when(jn < n_chunks)
                def _():
                    pltpu.async_copy(z_hbm.at[src_v.at[jn]], rows_v.at[t],
                                     gsem.at[t])
            return carry

        lax.fori_loop(0, n_groups, body, 0)
        plsc.subcore_barrier()

        pltpu.sync_copy(acc_sh.at[pl.ds(sid * ROWS_PER_TILE, ROWS_PER_TILE), :],
                        out_hbm.at[cid, pl.ds(sid * ROWS_PER_TILE, ROWS_PER_TILE), :])

    return sc_agg


_C1 = 80
_C2 = 80
_sc_agg1 = _make_sc_agg(D_HID, _C1)
_sc_agg2 = _make_sc_agg(D_OUT, _C2)

_CD = 80
_ND_CHUNKS = E_PER_TILE // _CD
_ND_GROUPS = _ND_CHUNKS // NBUF


@functools.partial(
    pl.kernel,
    mesh=_sc_mesh,
    compiler_params=_sc_params,
    out_type=jax.ShapeDtypeStruct((NC, N_PAD, DDEG), jnp.float32),
    scratch_types=[
        pltpu.VMEM((_ND_CHUNKS, _CD), jnp.int32),
        pltpu.VMEM((_CD, DDEG), jnp.float32),
        pltpu.VMEM_SHARED((N_PAD, DDEG), jnp.float32),
        pltpu.SemaphoreType.DMA((NBUF,)),
    ],
)
def _sc_deg(dst_hbm, ones_hbm, zinit_hbm, out_hbm, dst_v, ones_v, acc_sh, ssem):
    cid = lax.axis_index("c")
    sid = lax.axis_index("s")
    wid = cid * NS + sid

    pltpu.sync_copy(dst_hbm.at[wid], dst_v)
    pltpu.sync_copy(ones_hbm, ones_v)
    pltpu.sync_copy(zinit_hbm,
                    acc_sh.at[pl.ds(sid * ROWS_PER_TILE, ROWS_PER_TILE), :])
    plsc.subcore_barrier()

    def body(g, carry):
        j0 = g * NBUF
        for t in range(NBUF):
            pltpu.async_copy(ones_v, acc_sh.at[dst_v.at[j0 + t]],
                             ssem.at[t], add=True)
        for t in range(NBUF):
            pltpu.make_async_copy(ones_v, acc_sh.at[dst_v.at[j0 + t]],
                                  ssem.at[t]).wait()
        return carry

    lax.fori_loop(0, _ND_GROUPS, body, 0)
    plsc.subcore_barrier()

    pltpu.sync_copy(acc_sh.at[pl.ds(sid * ROWS_PER_TILE, ROWS_PER_TILE), :],
                    out_hbm.at[cid, pl.ds(sid * ROWS_PER_TILE, ROWS_PER_TILE), :])


_RB = 1000
_NRB = N_NODES // _RB


def _tc1_body(x_ref, w1l_ref, w1r_ref, b1r_ref, z1_ref, r1_ref):
    x = x_ref[...]
    z1 = jnp.dot(x, w1l_ref[...], preferred_element_type=jnp.float32)
    z1_ref[...] = z1.astype(jnp.bfloat16)
    r1_ref[...] = (jnp.dot(x, w1r_ref[...], preferred_element_type=jnp.float32)
                   + b1r_ref[...])


def _tc1(x, w1l, w1r, b1r):
    return pl.pallas_call(
        _tc1_body,
        grid=(_NRB,),
        in_specs=[
            pl.BlockSpec((_RB, D_IN), lambda i: (i, 0)),
            pl.BlockSpec((D_IN, D_HID), lambda i: (0, 0)),
            pl.BlockSpec((D_IN, D_HID), lambda i: (0, 0)),
            pl.BlockSpec((1, D_HID), lambda i: (0, 0)),
        ],
        out_specs=[
            pl.BlockSpec((_RB, D_HID), lambda i: (i, 0)),
            pl.BlockSpec((_RB, D_HID), lambda i: (i, 0)),
        ],
        out_shape=[
            jax.ShapeDtypeStruct((N_NODES, D_HID), jnp.bfloat16),
            jax.ShapeDtypeStruct((N_NODES, D_HID), jnp.float32),
        ],
    )(x, w1l, w1r, b1r)


def _tc2_body(p1_ref, ds_ref, r1_ref, b1l_ref, w2l_ref, w2r_ref, b2c_ref,
              z2_ref, r2p_ref):
    agg = (p1_ref[0].astype(jnp.float32)
           + p1_ref[1].astype(jnp.float32))
    ds = ds_ref[0] + ds_ref[1]
    deg = ds[:, 0:1]
    invd = 1.0 / jnp.maximum(deg, 1.0)
    h = jnp.maximum(agg * invd + b1l_ref[...] + r1_ref[...], 0.0)
    z2 = jnp.dot(h, w2l_ref[...], preferred_element_type=jnp.float32)
    z2_ref[...] = z2.astype(jnp.bfloat16)
    r2c = (jnp.dot(h, w2r_ref[...], preferred_element_type=jnp.float32)
           + b2c_ref[...])
    pad = jnp.zeros((_RB, D2P - D_OUT - 1), jnp.float32)
    r2p_ref[...] = jnp.concatenate([r2c, invd, pad], axis=1)


def _tc2(p1, dsum, r1, b1l, w2l, w2r, b2c):
    return pl.pallas_call(
        _tc2_body,
        grid=(_NRB,),
        in_specs=[
            pl.BlockSpec((NC, _RB, D_HID), lambda i: (0, i, 0)),
            pl.BlockSpec((NC, _RB, DDEG), lambda i: (0, i, 0)),
            pl.BlockSpec((_RB, D_HID), lambda i: (i, 0)),
            pl.BlockSpec((1, D_HID), lambda i: (0, 0)),
            pl.BlockSpec((D_HID, D_OUT), lambda i: (0, 0)),
            pl.BlockSpec((D_HID, D_OUT), lambda i: (0, 0)),
            pl.BlockSpec((1, D_OUT), lambda i: (0, 0)),
        ],
        out_specs=[
            pl.BlockSpec((_RB, D_OUT), lambda i: (i, 0)),
            pl.BlockSpec((_RB, D2P), lambda i: (i, 0)),
        ],
        out_shape=[
            jax.ShapeDtypeStruct((N_NODES, D_OUT), jnp.bfloat16),
            jax.ShapeDtypeStruct((N_NODES, D2P), jnp.float32),
        ],
    )(p1, dsum, r1, b1l, w2l, w2r, b2c)


def _tc3_body(p2_ref, r2p_ref, y_ref, m_ref, out_ref, num_ref, den_ref):
    i = pl.program_id(0)

    agg2 = (p2_ref[0].astype(jnp.float32)
            + p2_ref[1].astype(jnp.float32))
    r2c = r2p_ref[:, :D_OUT]
    invd = r2p_ref[:, D_OUT:D_OUT + 1]
    logits = agg2 * invd + r2c
    mx = jnp.max(logits, axis=1, keepdims=True)
    lse = jnp.log(jnp.sum(jnp.exp(logits - mx), axis=1, keepdims=True))
    lsm = logits - mx - lse
    onehot = (lax.broadcasted_iota(jnp.int32, (_RB, D_OUT), 1)
              == y_ref[...]).astype(jnp.float32)
    picked = jnp.sum(lsm * onehot, axis=1, keepdims=True)
    m = m_ref[...]
    num_p = jnp.sum(picked * m)
    den_p = jnp.sum(m)

    @pl.when(i == 0)
    def _():
        num_ref[0] = num_p
        den_ref[0] = den_p

    @pl.when(i > 0)
    def _():
        num_ref[0] = num_ref[0] + num_p
        den_ref[0] = den_ref[0] + den_p

    @pl.when(i == _NRB - 1)
    def _():
        loss = -num_ref[0] / jnp.maximum(den_ref[0], 1.0)
        out_ref[...] = jnp.broadcast_to(loss, (1, 1))


def _tc3(p2, r2p, y2d, m2d):
    return pl.pallas_call(
        _tc3_body,
        grid=(_NRB,),
        in_specs=[
            pl.BlockSpec((NC, _RB, D_OUT), lambda i: (0, i, 0)),
            pl.BlockSpec((_RB, D2P), lambda i: (i, 0)),
            pl.BlockSpec((_RB, 1), lambda i: (i, 0)),
            pl.BlockSpec((_RB, 1), lambda i: (i, 0)),
        ],
        out_specs=pl.BlockSpec((1, 1), lambda i: (0, 0)),
        out_shape=jax.ShapeDtypeStruct((1, 1), jnp.float32),
        scratch_shapes=[
            pltpu.SMEM((1,), jnp.float32),
            pltpu.SMEM((1,), jnp.float32),
        ],
    )(p2, r2p, y2d, m2d)


def kernel(x, edge_index, y, train_mask,
           W1_l, b1_l, W1_r, b1_r, W2_l, b2_l, W2_r, b2_r):
    src = edge_index[0]
    dst = edge_index[1]
    src1 = src.reshape(NW, E_PER_TILE // _C1, _C1)
    dst1 = dst.reshape(NW, E_PER_TILE // _C1, _C1)
    src2 = src.reshape(NW, E_PER_TILE // _C2, _C2)
    dst2 = dst.reshape(NW, E_PER_TILE // _C2, _C2)
    zinit1 = jnp.zeros((ROWS_PER_TILE, D_HID), jnp.bfloat16)
    zinit2 = jnp.zeros((ROWS_PER_TILE, D_OUT), jnp.bfloat16)
    zinitd = jnp.zeros((ROWS_PER_TILE, DDEG), jnp.float32)
    onesd = jnp.ones((_CD, DDEG), jnp.float32)

    dsum = _sc_deg(dst2, onesd, zinitd)
    z1, r1 = _tc1(x, W1_l, W1_r, b1_r.reshape(1, D_HID))
    p1 = _sc_agg1(z1, src1, dst1, zinit1)
    b2c = (b2_l + b2_r).reshape(1, D_OUT)
    z2, r2p = _tc2(p1, dsum, r1, b1_l.reshape(1, D_HID), W2_l, W2_r, b2c)
    p2 = _sc_agg2(z2, src2, dst2, zinit2)
    loss = _tc3(p2, r2p, y.reshape(N_NODES, 1).astype(jnp.int32),
                train_mask.reshape(N_NODES, 1).astype(jnp.float32))
    return loss.reshape(1)
